# baseline (speedup 1.0000x reference)
import functools

import jax
import jax.numpy as jnp
from jax import lax
from jax.experimental import pallas as pl
from jax.experimental.pallas import tpu as pltpu
from jax.experimental.pallas import tpu_sc as plsc

_N_EMBED = 64
_SCALE = float(_N_EMBED) ** 0.5
_LANES = 16

_CHUNK = 128
_B = 16384 * 50
_NW = 32
_ROWS_PER_W = _B // _NW
_CHUNKS_PER_W = _ROWS_PER_W // _CHUNK


def _embed_body(x_hbm, table_hbm, out_hbm, idx_v, rows_v, gsem):
    nc = 2
    wid = lax.axis_index("s") * nc + lax.axis_index("c")

    pltpu.sync_copy(x_hbm.at[pl.ds(wid * _CHUNKS_PER_W, _CHUNKS_PER_W)], idx_v)

    row_base = wid * _ROWS_PER_W

    def chunk(g, _):
        pltpu.async_copy(table_hbm.at[idx_v.at[g]], rows_v, gsem).wait()

        def scale_row(r, _):
            for c in range(_N_EMBED // _LANES):
                sl = pl.ds(c * _LANES, _LANES)
                rows_v[r, sl] = rows_v[r, sl] * _SCALE
            return 0

        lax.fori_loop(0, _CHUNK, scale_row, 0)

        pltpu.sync_copy(rows_v, out_hbm.at[pl.ds(row_base + g * _CHUNK, _CHUNK)])
        return 0

    lax.fori_loop(0, _CHUNKS_PER_W, chunk, 0)


@jax.jit
def _embed(x_flat, embed_weight):
    mesh = plsc.VectorSubcoreMesh(core_axis_name="c", subcore_axis_name="s")
    run = pl.kernel(
        _embed_body,
        out_type=jax.ShapeDtypeStruct((_B, _N_EMBED), jnp.float32),
        mesh=mesh,
        scratch_types=[
            pltpu.VMEM((_CHUNKS_PER_W, _CHUNK), jnp.int32),
            pltpu.VMEM((_CHUNK, _N_EMBED), jnp.float32),
            pltpu.SemaphoreType.DMA,
        ],
        compiler_params=pltpu.CompilerParams(use_tc_tiling_on_sc=False),
    )
    return run(x_flat, embed_weight)


def kernel(x, embed_weight):
    x_flat = x.reshape(_B // _CHUNK, _CHUNK).astype(jnp.int32)
    out = _embed(x_flat, embed_weight)
    return out.reshape(x.shape[0], x.shape[1], _N_EMBED)


# speedup vs baseline: 1.1848x; 1.1848x over previous
import jax
import jax.numpy as jnp
from jax import lax
from jax.experimental import pallas as pl
from jax.experimental.pallas import tpu as pltpu
from jax.experimental.pallas import tpu_sc as plsc

_N_EMBED = 64
_SCALE = float(_N_EMBED) ** 0.5
_LANES = 16

_CHUNK = 128
_B = 16384 * 50
_NW = 32
_ROWS_PER_W = _B // _NW
_CHUNKS_PER_W = _ROWS_PER_W // _CHUNK
_NBUF = 4
_LOOKAHEAD = 2


def _embed_body(x_hbm, table_hbm, out_hbm, idx_v, rows_v, gsems, ssems):
    nc = 2
    wid = lax.axis_index("s") * nc + lax.axis_index("c")

    pltpu.sync_copy(x_hbm.at[pl.ds(wid * _CHUNKS_PER_W, _CHUNKS_PER_W)], idx_v)

    row_base = wid * _ROWS_PER_W

    def start_gather(g, b):
        pltpu.make_async_copy(
            table_hbm.at[idx_v.at[g]], rows_v.at[b], gsems.at[b]
        ).start()

    def wait_gather(b):
        pltpu.make_async_copy(
            table_hbm.at[idx_v.at[0]], rows_v.at[b], gsems.at[b]
        ).wait()

    def start_store(g, b):
        pltpu.make_async_copy(
            rows_v.at[b], out_hbm.at[pl.ds(row_base + g * _CHUNK, _CHUNK)],
            ssems.at[b],
        ).start()

    def wait_store(b):
        pltpu.make_async_copy(
            rows_v.at[b], out_hbm.at[pl.ds(row_base, _CHUNK)], ssems.at[b]
        ).wait()

    def scale(b):
        def body(i, _):
            r = i * 4
            for k in range(4):
                for c in range(_N_EMBED // _LANES):
                    sl = pl.ds(c * _LANES, _LANES)
                    rows_v[b, r + k, sl] = rows_v[b, r + k, sl] * _SCALE
            return 0

        lax.fori_loop(0, _CHUNK // 4, body, 0)

    for g in range(_LOOKAHEAD):
        start_gather(g, g % _NBUF)


    def emit(j, b, g, need_store_wait):
        wait_gather(b)
        scale(b)
        start_store(j, b)
        if g is not None:
            b2 = (b + _LOOKAHEAD) % _NBUF
            if need_store_wait:
                wait_store(b2)
            start_gather(g, b2)

    for j in range(_NBUF):
        emit(j, j % _NBUF, j + _LOOKAHEAD, j + _LOOKAHEAD >= _NBUF)

    n_steady = _CHUNKS_PER_W - _NBUF - _LOOKAHEAD
    n_groups = n_steady // _NBUF

    def steady(t, _):
        j0 = _NBUF + t * _NBUF
        for i in range(_NBUF):
            emit(j0 + i, i, j0 + i + _LOOKAHEAD, True)
        return 0

    lax.fori_loop(0, n_groups, steady, 0)

    for j in range(_NBUF + n_groups * _NBUF, _CHUNKS_PER_W):
        g = j + _LOOKAHEAD
        emit(j, j % _NBUF, g if g < _CHUNKS_PER_W else None, True)

    for b in range(_NBUF):
        wait_store(b)


@jax.jit
def _embed(x_flat, embed_weight):
    mesh = plsc.VectorSubcoreMesh(core_axis_name="c", subcore_axis_name="s")
    run = pl.kernel(
        _embed_body,
        out_type=jax.ShapeDtypeStruct((_B, _N_EMBED), jnp.float32),
        mesh=mesh,
        scratch_types=[
            pltpu.VMEM((_CHUNKS_PER_W, _CHUNK), jnp.int32),
            pltpu.VMEM((_NBUF, _CHUNK, _N_EMBED), jnp.float32),
            pltpu.SemaphoreType.DMA((_NBUF,)),
            pltpu.SemaphoreType.DMA((_NBUF,)),
        ],
        compiler_params=pltpu.CompilerParams(use_tc_tiling_on_sc=False),
    )
    return run(x_flat, embed_weight)


def kernel(x, embed_weight):
    x_flat = x.reshape(_B // _CHUNK, _CHUNK).astype(jnp.int32)
    out = _embed(x_flat, embed_weight)
    return out.reshape(x.shape[0], x.shape[1], _N_EMBED)
